# baseline (speedup 1.0000x reference)
import functools

import jax
import jax.numpy as jnp
from jax import lax
from jax.experimental import pallas as pl
from jax.experimental.pallas import tpu as pltpu
from jax.experimental.pallas import tpu_sc as plsc

N_NEWS = 10000
N_USERS = 10000
N_EDGES = 320000
D = 128
DEGW = 16

NC = 2
NS = 16
NW = NC * NS
CHUNK = 64
E_PAD = 327680
EPT = E_PAD // NW
NCHUNK = EPT // CHUNK
NU_PAD = 10240
ROWS_PT = NU_PAD // NS
ZROWS = CHUNK


def _sc_body(news_hbm, idx_hbm, agr_out, deg_out,
             idx0, idx1, idx2, idx3, gbuf0, gbuf1, ones_v, zdeg,
             agr_sh, deg_sh,
             sem_g0, sem_g1, sem_s, sem_d, sem_i0, sem_i1, sem_i2, sem_i3):
    c = lax.axis_index("c")
    s = lax.axis_index("s")
    wid = s * NC + c
    sem_g = (sem_g0, sem_g1)
    sem_i = (sem_i0, sem_i1, sem_i2, sem_i3)
    idx_r = (idx0, idx1, idx2, idx3)
    gbuf_b = (gbuf0, gbuf1)

    zeros16 = jnp.zeros((16,), jnp.float32)
    ones16 = jnp.ones((16,), jnp.float32)

    def zfill(i, _):
        r = i // 8
        col8 = (i % 8) * 16
        gbuf0[r, pl.ds(col8, 16)] = zeros16
        return 0
    lax.fori_loop(0, ZROWS * (D // 16), zfill, 0)

    def zdfill(i, _):
        zdeg[i, pl.ds(0, 16)] = zeros16
        ones_v[i, pl.ds(0, 16)] = ones16
        return 0
    lax.fori_loop(0, CHUNK, zdfill, 0)

    base = s * ROWS_PT
    for k in range(ROWS_PT // ZROWS):
        pltpu.sync_copy(gbuf0, agr_sh.at[pl.ds(base + k * ZROWS, ZROWS)])
        pltpu.sync_copy(zdeg, deg_sh.at[pl.ds(base + k * ZROWS, ZROWS)])

    def wait_rows(sem, dst):
        pltpu.make_async_copy(news_hbm.at[pl.ds(0, CHUNK)], dst, sem).wait()

    def wait_deg(sem):
        pltpu.make_async_copy(deg_out.at[0].at[pl.ds(0, CHUNK)], zdeg,
                              sem).wait()

    my_idx = idx_hbm.at[wid]
    for r in range(4):
        pltpu.async_copy(my_idx.at[r], idx_r[r], sem_i[r])

    plsc.subcore_barrier()

    def stage(j, r, b):
        gbuf = gbuf_b[b]
        idxr = idx_r[r]
        pltpu.make_async_copy(my_idx.at[j], idxr, sem_i[r]).wait()
        pltpu.async_copy(news_hbm.at[idxr.at[0]], gbuf, sem_g[b])
        wait_rows(sem_g[b], gbuf)
        pltpu.sync_copy(gbuf, agr_sh.at[idxr.at[1]], add=True)
        pltpu.sync_copy(ones_v, deg_sh.at[idxr.at[1]], add=True)

        @pl.when(j + 4 < NCHUNK)
        def _():
            pltpu.async_copy(my_idx.at[j + 4], idx_r[r], sem_i[r])

    def step(t, _):
        j0 = 4 * t
        stage(j0 + 0, 0, 0)
        stage(j0 + 1, 1, 1)
        stage(j0 + 2, 2, 0)
        stage(j0 + 3, 3, 1)
        return 0
    lax.fori_loop(0, NCHUNK // 4, step, 0)

    plsc.subcore_barrier()

    for k in range(ROWS_PT // ZROWS):
        sl = pl.ds(base + k * ZROWS, ZROWS)
        pltpu.sync_copy(agr_sh.at[sl], agr_out.at[c].at[sl])
        pltpu.sync_copy(deg_sh.at[sl], deg_out.at[c].at[sl])


_sc_call = functools.partial(
    pl.kernel,
    out_type=[
        jax.ShapeDtypeStruct((NC, NU_PAD, D), jnp.float32),
        jax.ShapeDtypeStruct((NC, NU_PAD, DEGW), jnp.float32),
    ],
    mesh=plsc.VectorSubcoreMesh(core_axis_name="c", subcore_axis_name="s",
                                num_cores=NC, num_subcores=NS),
    scratch_types=[
        pltpu.VMEM((2, CHUNK), jnp.int32),
        pltpu.VMEM((2, CHUNK), jnp.int32),
        pltpu.VMEM((2, CHUNK), jnp.int32),
        pltpu.VMEM((2, CHUNK), jnp.int32),
        pltpu.VMEM((CHUNK, D), jnp.float32),
        pltpu.VMEM((CHUNK, D), jnp.float32),
        pltpu.VMEM((CHUNK, DEGW), jnp.float32),
        pltpu.VMEM((CHUNK, DEGW), jnp.float32),
        pltpu.VMEM_SHARED((NU_PAD, D), jnp.float32),
        pltpu.VMEM_SHARED((NU_PAD, DEGW), jnp.float32),
        pltpu.SemaphoreType.DMA,
        pltpu.SemaphoreType.DMA,
        pltpu.SemaphoreType.DMA,
        pltpu.SemaphoreType.DMA,
        pltpu.SemaphoreType.DMA,
        pltpu.SemaphoreType.DMA,
        pltpu.SemaphoreType.DMA,
        pltpu.SemaphoreType.DMA,
    ],
    compiler_params=pltpu.CompilerParams(use_tc_tiling_on_sc=False),
)(_sc_body)


BLK = 1024


def _mlp_body(user_ref, agrp_ref, degp_ref, w1u_ref, w1a_ref, b1_ref,
              w2_ref, b2_ref, out_ref):
    agr = agrp_ref[0] + agrp_ref[1]
    deg = degp_ref[0, :, 0:1] + degp_ref[1, :, 0:1]
    agr = agr / (deg + 1e-8)
    h = jnp.tanh(
        jnp.dot(user_ref[...], w1u_ref[...], preferred_element_type=jnp.float32)
        + jnp.dot(agr, w1a_ref[...], preferred_element_type=jnp.float32)
        + b1_ref[...])
    out_ref[...] = (
        jnp.dot(h, w2_ref[...], preferred_element_type=jnp.float32)
        + b2_ref[...])


def _mlp_call(user_feats, agr_p, deg_p, w1u, w1a, b1, w2, b2):
    grid = (NU_PAD // BLK,)
    return pl.pallas_call(
        _mlp_body,
        grid=grid,
        in_specs=[
            pl.BlockSpec((BLK, D), lambda i: (i, 0)),
            pl.BlockSpec((NC, BLK, D), lambda i: (0, i, 0)),
            pl.BlockSpec((NC, BLK, DEGW), lambda i: (0, i, 0)),
            pl.BlockSpec((D, D), lambda i: (0, 0)),
            pl.BlockSpec((D, D), lambda i: (0, 0)),
            pl.BlockSpec((1, D), lambda i: (0, 0)),
            pl.BlockSpec((D, D), lambda i: (0, 0)),
            pl.BlockSpec((1, D), lambda i: (0, 0)),
        ],
        out_specs=pl.BlockSpec((BLK, D), lambda i: (i, 0)),
        out_shape=jax.ShapeDtypeStruct((NU_PAD, D), jnp.float32),
    )(user_feats, agr_p, deg_p, w1u, w1a, b1, w2, b2)


def kernel(news_feats, user_feats, edge_index, W1, b1, W2, b2):
    pad_n = E_PAD - N_EDGES
    row = jnp.concatenate(
        [edge_index[0].astype(jnp.int32), jnp.zeros((pad_n,), jnp.int32)])
    col = jnp.concatenate(
        [edge_index[1].astype(jnp.int32),
         N_USERS + (jnp.arange(pad_n, dtype=jnp.int32) % (NU_PAD - N_USERS))])
    idx = jnp.stack([row.reshape(NW, NCHUNK, CHUNK),
                     col.reshape(NW, NCHUNK, CHUNK)], axis=2)
    agr_p, deg_p = _sc_call(news_feats, idx)
    w1u = W1[:, :D].T
    w1a = W1[:, D:].T
    w2 = W2.T
    user_pad = jnp.pad(user_feats, ((0, NU_PAD - N_USERS), (0, 0)))
    out = _mlp_call(user_pad, agr_p, deg_p, w1u, w1a,
                    b1.reshape(1, D), w2, b2.reshape(1, D))
    return out[:N_USERS]


# speedup vs baseline: 2.3635x; 2.3635x over previous
import functools

import jax
import jax.numpy as jnp
from jax import lax
from jax.experimental import pallas as pl
from jax.experimental.pallas import tpu as pltpu
from jax.experimental.pallas import tpu_sc as plsc

N_NEWS = 10000
N_USERS = 10000
N_EDGES = 320000
D = 128
DEGW = 16

NC = 2
NS = 16
NW = NC * NS
EPT = N_EDGES // NW
CHUNK = 80
NCHUNK = EPT // CHUNK
NU_PAD = 10240
ROWS_PT = NU_PAD // NS


def _sc_body(news_hbm, row_hbm, col_hbm, agr_out, deg_out,
             row_v, col_v, gbuf, ones_v, zdeg, agr_sh, deg_sh, sem, sem_d):
    c = lax.axis_index("c")
    s = lax.axis_index("s")
    wid = s * NC + c

    zeros16 = jnp.zeros((16,), jnp.float32)
    ones16 = jnp.ones((16,), jnp.float32)

    def zfill(i, _):
        r = i // 8
        col8 = (i % 8) * 16
        gbuf[r, pl.ds(col8, 16)] = zeros16
        return 0
    lax.fori_loop(0, CHUNK * (D // 16), zfill, 0)

    def zdfill(i, _):
        zdeg[i, pl.ds(0, 16)] = zeros16
        ones_v[i, pl.ds(0, 16)] = ones16
        return 0
    lax.fori_loop(0, CHUNK, zdfill, 0)

    base = s * ROWS_PT
    for k in range(ROWS_PT // CHUNK):
        pltpu.sync_copy(gbuf, agr_sh.at[pl.ds(base + k * CHUNK, CHUNK)])
        pltpu.sync_copy(zdeg, deg_sh.at[pl.ds(base + k * CHUNK, CHUNK)])

    pltpu.sync_copy(row_hbm.at[wid], row_v)
    pltpu.sync_copy(col_hbm.at[wid], col_v)

    plsc.subcore_barrier()

    def step(j, _):
        pltpu.async_copy(news_hbm.at[row_v.at[j]], gbuf, sem).wait()
        pltpu.async_copy(ones_v, deg_sh.at[col_v.at[j]], sem_d, add=True)
        pltpu.sync_copy(gbuf, agr_sh.at[col_v.at[j]], add=True)
        return 0
    lax.fori_loop(0, NCHUNK, step, 0)

    def drain(j, _):
        pltpu.make_async_copy(deg_out.at[0].at[pl.ds(0, CHUNK)], zdeg,
                              sem_d).wait()
        return 0
    lax.fori_loop(0, NCHUNK, drain, 0)

    plsc.subcore_barrier()

    for k in range(ROWS_PT // CHUNK):
        sl = pl.ds(base + k * CHUNK, CHUNK)
        pltpu.sync_copy(agr_sh.at[sl], agr_out.at[c].at[sl])
        pltpu.sync_copy(deg_sh.at[sl], deg_out.at[c].at[sl])


_sc_call = functools.partial(
    pl.kernel,
    out_type=[
        jax.ShapeDtypeStruct((NC, NU_PAD, D), jnp.float32),
        jax.ShapeDtypeStruct((NC, NU_PAD, DEGW), jnp.float32),
    ],
    mesh=plsc.VectorSubcoreMesh(core_axis_name="c", subcore_axis_name="s",
                                num_cores=NC, num_subcores=NS),
    scratch_types=[
        pltpu.VMEM((NCHUNK, CHUNK), jnp.int32),
        pltpu.VMEM((NCHUNK, CHUNK), jnp.int32),
        pltpu.VMEM((CHUNK, D), jnp.float32),
        pltpu.VMEM((CHUNK, DEGW), jnp.float32),
        pltpu.VMEM((CHUNK, DEGW), jnp.float32),
        pltpu.VMEM_SHARED((NU_PAD, D), jnp.float32),
        pltpu.VMEM_SHARED((NU_PAD, DEGW), jnp.float32),
        pltpu.SemaphoreType.DMA,
        pltpu.SemaphoreType.DMA,
    ],
    compiler_params=pltpu.CompilerParams(use_tc_tiling_on_sc=False),
)(_sc_body)


BLK = 1024


def _mlp_body(user_ref, agrp_ref, degp_ref, w1u_ref, w1a_ref, b1_ref,
              w2_ref, b2_ref, out_ref):
    agr = agrp_ref[0] + agrp_ref[1]
    deg = degp_ref[0, :, 0:1] + degp_ref[1, :, 0:1]
    agr = agr / (deg + 1e-8)
    h = jnp.tanh(
        jnp.dot(user_ref[...], w1u_ref[...], preferred_element_type=jnp.float32)
        + jnp.dot(agr, w1a_ref[...], preferred_element_type=jnp.float32)
        + b1_ref[...])
    out_ref[...] = (
        jnp.dot(h, w2_ref[...], preferred_element_type=jnp.float32)
        + b2_ref[...])


def _mlp_call(user_feats, agr_p, deg_p, w1u, w1a, b1, w2, b2):
    grid = (NU_PAD // BLK,)
    return pl.pallas_call(
        _mlp_body,
        grid=grid,
        in_specs=[
            pl.BlockSpec((BLK, D), lambda i: (i, 0)),
            pl.BlockSpec((NC, BLK, D), lambda i: (0, i, 0)),
            pl.BlockSpec((NC, BLK, DEGW), lambda i: (0, i, 0)),
            pl.BlockSpec((D, D), lambda i: (0, 0)),
            pl.BlockSpec((D, D), lambda i: (0, 0)),
            pl.BlockSpec((1, D), lambda i: (0, 0)),
            pl.BlockSpec((D, D), lambda i: (0, 0)),
            pl.BlockSpec((1, D), lambda i: (0, 0)),
        ],
        out_specs=pl.BlockSpec((BLK, D), lambda i: (i, 0)),
        out_shape=jax.ShapeDtypeStruct((NU_PAD, D), jnp.float32),
    )(user_feats, agr_p, deg_p, w1u, w1a, b1, w2, b2)


def kernel(news_feats, user_feats, edge_index, W1, b1, W2, b2):
    row = edge_index[0].astype(jnp.int32).reshape(NW, NCHUNK, CHUNK)
    col = edge_index[1].astype(jnp.int32).reshape(NW, NCHUNK, CHUNK)
    agr_p, deg_p = _sc_call(news_feats, row, col)
    w1u = W1[:, :D].T
    w1a = W1[:, D:].T
    w2 = W2.T
    user_pad = jnp.pad(user_feats, ((0, NU_PAD - N_USERS), (0, 0)))
    out = _mlp_call(user_pad, agr_p, deg_p, w1u, w1a,
                    b1.reshape(1, D), w2, b2.reshape(1, D))
    return out[:N_USERS]
